# baseline (speedup 1.0000x reference)
import functools

import jax
import jax.numpy as jnp
from jax.experimental import pallas as pl
from jax.experimental.pallas import tpu as pltpu

_IGNORE = -100


def _fused_tagger_kernel(emb_ref, labels_ref, w_ih_ref, b_ref, w_hh_ref,
                         w_cls_ref, b_cls_ref, total_ref, count_ref,
                         h_sc, c_sc, hbuf_sc, *, t_blk, b_blk, hdim):
    @pl.when(pl.program_id(1) == 0)
    def _init():
        h_sc[...] = jnp.zeros_like(h_sc)
        c_sc[...] = jnp.zeros_like(c_sc)
        total_ref[...] = jnp.zeros_like(total_ref)
        count_ref[...] = jnp.zeros_like(count_ref)

    gx = jnp.dot(emb_ref[0], w_ih_ref[...],
                 preferred_element_type=jnp.float32) + b_ref[...]

    whh = w_hh_ref[...]
    for t in range(t_blk):
        gates = gx[t * b_blk:(t + 1) * b_blk] + jnp.dot(
            h_sc[...].astype(jnp.bfloat16), whh,
            preferred_element_type=jnp.float32)
        i_g = jax.nn.sigmoid(gates[:, 0 * hdim:1 * hdim])
        f_g = jax.nn.sigmoid(gates[:, 1 * hdim:2 * hdim])
        g_g = jnp.tanh(gates[:, 2 * hdim:3 * hdim])
        o_g = jax.nn.sigmoid(gates[:, 3 * hdim:4 * hdim])
        c_new = f_g * c_sc[...] + i_g * g_g
        h_new = o_g * jnp.tanh(c_new)
        c_sc[...] = c_new
        h_sc[...] = h_new
        hbuf_sc[t * b_blk:(t + 1) * b_blk, :] = h_new.astype(jnp.bfloat16)

    logits = jnp.dot(hbuf_sc[...], w_cls_ref[...],
                     preferred_element_type=jnp.float32) + b_cls_ref[...]
    lab = labels_ref[0, 0]
    valid = lab != _IGNORE
    m = jnp.max(logits, axis=1, keepdims=True)
    lse = m + jnp.log(jnp.sum(jnp.exp(logits - m), axis=1, keepdims=True))
    cls = jax.lax.broadcasted_iota(jnp.int32, logits.shape, 1)
    safe = jnp.where(valid, lab, 0)
    picked = jnp.sum(jnp.where(cls == safe, logits, 0.0), axis=1,
                     keepdims=True)
    nll = jnp.where(valid, lse - picked, 0.0)
    total_ref[...] += jnp.sum(nll).reshape(1, 1, 1)
    count_ref[...] += jnp.sum(valid.astype(jnp.float32)).reshape(1, 1, 1)


def kernel(tokens, labels, embedding, w_ih_t, w_hh_t, b, w_cls_t, b_cls):
    B, T = tokens.shape
    E = embedding.shape[1]
    H = w_hh_t.shape[0]
    C = w_cls_t.shape[1]

    b_blk = 16 if B % 16 == 0 else B
    n_sh = B // b_blk
    t_blk = 8 if T % 8 == 0 else T
    n_tb = T // t_blk

    rows = t_blk * b_blk
    tokens_r = tokens.reshape(n_sh, b_blk, T).transpose(0, 2, 1) \
                     .reshape(n_sh, T * b_blk)
    emb_r = embedding[tokens_r].astype(jnp.bfloat16)
    labels_r = labels.reshape(n_sh, b_blk, n_tb, t_blk) \
                     .transpose(0, 2, 3, 1).reshape(n_sh, n_tb, rows, 1)

    total, count = pl.pallas_call(
        functools.partial(_fused_tagger_kernel, t_blk=t_blk, b_blk=b_blk,
                          hdim=H),
        out_shape=(jax.ShapeDtypeStruct((n_sh, 1, 1), jnp.float32),
                   jax.ShapeDtypeStruct((n_sh, 1, 1), jnp.float32)),
        grid_spec=pltpu.PrefetchScalarGridSpec(
            num_scalar_prefetch=0,
            grid=(n_sh, n_tb),
            in_specs=[
                pl.BlockSpec((1, rows, E), lambda s, t: (s, t, 0)),
                pl.BlockSpec((1, 1, rows, 1), lambda s, t: (s, t, 0, 0)),
                pl.BlockSpec((E, 4 * H), lambda s, t: (0, 0)),
                pl.BlockSpec((1, 4 * H), lambda s, t: (0, 0)),
                pl.BlockSpec((H, 4 * H), lambda s, t: (0, 0)),
                pl.BlockSpec((H, C), lambda s, t: (0, 0)),
                pl.BlockSpec((1, C), lambda s, t: (0, 0)),
            ],
            out_specs=[
                pl.BlockSpec((1, 1, 1), lambda s, t: (s, 0, 0)),
                pl.BlockSpec((1, 1, 1), lambda s, t: (s, 0, 0)),
            ],
            scratch_shapes=[
                pltpu.VMEM((b_blk, H), jnp.float32),
                pltpu.VMEM((b_blk, H), jnp.float32),
                pltpu.VMEM((t_blk * b_blk, H), jnp.bfloat16),
            ],
        ),
        compiler_params=pltpu.CompilerParams(
            dimension_semantics=("parallel", "arbitrary"),
            vmem_limit_bytes=64 * 1024 * 1024),
    )(emb_r, labels_r, w_ih_t.astype(jnp.bfloat16), b,
      w_hh_t.astype(jnp.bfloat16), w_cls_t.astype(jnp.bfloat16), b_cls)

    return jnp.sum(total) / jnp.sum(count)


# speedup vs baseline: 1.4490x; 1.4490x over previous
import functools

import jax
import jax.numpy as jnp
from jax.experimental import pallas as pl
from jax.experimental.pallas import tpu as pltpu

_IGNORE = -100


def _fused_tagger_kernel(emb_ref, labels_ref, w_ih_ref, b_ref, w_hh_ref,
                         w_cls_ref, b_cls_ref, total_ref, count_ref,
                         h_sc, c_sc, hbuf_sc, *, t_blk, b_blk, hdim):
    @pl.when(pl.program_id(1) == 0)
    def _init():
        h_sc[...] = jnp.zeros_like(h_sc)
        c_sc[...] = jnp.zeros_like(c_sc)
        total_ref[...] = jnp.zeros_like(total_ref)
        count_ref[...] = jnp.zeros_like(count_ref)

    gx = jnp.dot(emb_ref[0], w_ih_ref[...],
                 preferred_element_type=jnp.float32) + b_ref[...]

    whh = w_hh_ref[...]
    for t in range(t_blk):
        gates = gx[t * b_blk:(t + 1) * b_blk] + jnp.dot(
            h_sc[...].astype(jnp.bfloat16), whh,
            preferred_element_type=jnp.float32)
        i_g = jax.nn.sigmoid(gates[:, 0 * hdim:1 * hdim])
        f_g = jax.nn.sigmoid(gates[:, 1 * hdim:2 * hdim])
        g_g = jnp.tanh(gates[:, 2 * hdim:3 * hdim])
        o_g = jax.nn.sigmoid(gates[:, 3 * hdim:4 * hdim])
        c_new = f_g * c_sc[...] + i_g * g_g
        h_new = o_g * jnp.tanh(c_new)
        c_sc[...] = c_new
        h_sc[...] = h_new
        hbuf_sc[t * b_blk:(t + 1) * b_blk, :] = h_new.astype(jnp.bfloat16)

    logits = jnp.dot(hbuf_sc[...], w_cls_ref[...],
                     preferred_element_type=jnp.float32) + b_cls_ref[...]
    lab = labels_ref[0, 0]
    valid = lab != _IGNORE
    m = jnp.max(logits, axis=1, keepdims=True)
    lse = m + jnp.log(jnp.sum(jnp.exp(logits - m), axis=1, keepdims=True))
    cls = jax.lax.broadcasted_iota(jnp.int32, logits.shape, 1)
    safe = jnp.where(valid, lab, 0)
    picked = jnp.sum(jnp.where(cls == safe, logits, 0.0), axis=1,
                     keepdims=True)
    nll = jnp.where(valid, lse - picked, 0.0)
    total_ref[...] += jnp.sum(nll).reshape(1, 1, 1)
    count_ref[...] += jnp.sum(valid.astype(jnp.float32)).reshape(1, 1, 1)


def kernel(tokens, labels, embedding, w_ih_t, w_hh_t, b, w_cls_t, b_cls):
    B, T = tokens.shape
    E = embedding.shape[1]
    H = w_hh_t.shape[0]
    C = w_cls_t.shape[1]

    b_blk = B
    n_sh = B // b_blk
    t_blk = 8 if T % 8 == 0 else T
    n_tb = T // t_blk

    rows = t_blk * b_blk
    tokens_r = tokens.reshape(n_sh, b_blk, T).transpose(0, 2, 1) \
                     .reshape(n_sh, T * b_blk)
    emb_r = embedding[tokens_r].astype(jnp.bfloat16)
    labels_r = labels.reshape(n_sh, b_blk, n_tb, t_blk) \
                     .transpose(0, 2, 3, 1).reshape(n_sh, n_tb, rows, 1)

    total, count = pl.pallas_call(
        functools.partial(_fused_tagger_kernel, t_blk=t_blk, b_blk=b_blk,
                          hdim=H),
        out_shape=(jax.ShapeDtypeStruct((n_sh, 1, 1), jnp.float32),
                   jax.ShapeDtypeStruct((n_sh, 1, 1), jnp.float32)),
        grid_spec=pltpu.PrefetchScalarGridSpec(
            num_scalar_prefetch=0,
            grid=(n_sh, n_tb),
            in_specs=[
                pl.BlockSpec((1, rows, E), lambda s, t: (s, t, 0)),
                pl.BlockSpec((1, 1, rows, 1), lambda s, t: (s, t, 0, 0)),
                pl.BlockSpec((E, 4 * H), lambda s, t: (0, 0)),
                pl.BlockSpec((1, 4 * H), lambda s, t: (0, 0)),
                pl.BlockSpec((H, 4 * H), lambda s, t: (0, 0)),
                pl.BlockSpec((H, C), lambda s, t: (0, 0)),
                pl.BlockSpec((1, C), lambda s, t: (0, 0)),
            ],
            out_specs=[
                pl.BlockSpec((1, 1, 1), lambda s, t: (s, 0, 0)),
                pl.BlockSpec((1, 1, 1), lambda s, t: (s, 0, 0)),
            ],
            scratch_shapes=[
                pltpu.VMEM((b_blk, H), jnp.float32),
                pltpu.VMEM((b_blk, H), jnp.float32),
                pltpu.VMEM((t_blk * b_blk, H), jnp.bfloat16),
            ],
        ),
        compiler_params=pltpu.CompilerParams(
            dimension_semantics=("parallel", "arbitrary"),
            vmem_limit_bytes=64 * 1024 * 1024),
    )(emb_r, labels_r, w_ih_t.astype(jnp.bfloat16), b,
      w_hh_t.astype(jnp.bfloat16), w_cls_t.astype(jnp.bfloat16), b_cls)

    return jnp.sum(total) / jnp.sum(count)


# speedup vs baseline: 1.4975x; 1.0334x over previous
import functools

import jax
import jax.numpy as jnp
from jax.experimental import pallas as pl
from jax.experimental.pallas import tpu as pltpu

_IGNORE = -100


def _fused_tagger_kernel(emb_ref, labels_ref, w_ih_ref, b_ref, w_hh_ref,
                         w_cls_ref, b_cls_ref, total_ref, count_ref,
                         h_sc, c_sc, hbuf_sc, *, t_blk, b_blk, hdim):
    @pl.when(pl.program_id(1) == 0)
    def _init():
        h_sc[...] = jnp.zeros_like(h_sc)
        c_sc[...] = jnp.zeros_like(c_sc)
        total_ref[...] = jnp.zeros_like(total_ref)
        count_ref[...] = jnp.zeros_like(count_ref)

    gx = jnp.dot(emb_ref[0], w_ih_ref[...],
                 preferred_element_type=jnp.float32) + b_ref[...]

    whh = w_hh_ref[...]
    for t in range(t_blk):
        gates = gx[t * b_blk:(t + 1) * b_blk] + jnp.dot(
            h_sc[...].astype(jnp.bfloat16), whh,
            preferred_element_type=jnp.float32)
        i_g = jax.nn.sigmoid(gates[:, 0 * hdim:1 * hdim])
        f_g = jax.nn.sigmoid(gates[:, 1 * hdim:2 * hdim])
        g_g = jnp.tanh(gates[:, 2 * hdim:3 * hdim])
        o_g = jax.nn.sigmoid(gates[:, 3 * hdim:4 * hdim])
        c_new = f_g * c_sc[...] + i_g * g_g
        h_new = o_g * jnp.tanh(c_new)
        c_sc[...] = c_new
        h_sc[...] = h_new
        hbuf_sc[t * b_blk:(t + 1) * b_blk, :] = h_new.astype(jnp.bfloat16)

    logits = jnp.dot(hbuf_sc[...], w_cls_ref[...],
                     preferred_element_type=jnp.float32) + b_cls_ref[...]
    lab = labels_ref[0, 0]
    valid = lab != _IGNORE
    m = jnp.max(logits, axis=1, keepdims=True)
    lse = m + jnp.log(jnp.sum(jnp.exp(logits - m), axis=1, keepdims=True))
    cls = jax.lax.broadcasted_iota(jnp.int32, logits.shape, 1)
    safe = jnp.where(valid, lab, 0)
    picked = jnp.sum(jnp.where(cls == safe, logits, 0.0), axis=1,
                     keepdims=True)
    nll = jnp.where(valid, lse - picked, 0.0)
    total_ref[...] += jnp.sum(nll).reshape(1, 1, 1)
    count_ref[...] += jnp.sum(valid.astype(jnp.float32)).reshape(1, 1, 1)


def kernel(tokens, labels, embedding, w_ih_t, w_hh_t, b, w_cls_t, b_cls):
    B, T = tokens.shape
    E = embedding.shape[1]
    H = w_hh_t.shape[0]
    C = w_cls_t.shape[1]

    b_blk = B
    n_sh = B // b_blk
    t_blk = 16 if T % 16 == 0 else T
    n_tb = T // t_blk

    rows = t_blk * b_blk
    tokens_r = tokens.reshape(n_sh, b_blk, T).transpose(0, 2, 1) \
                     .reshape(n_sh, T * b_blk)
    emb_r = embedding[tokens_r].astype(jnp.bfloat16)
    labels_r = labels.reshape(n_sh, b_blk, n_tb, t_blk) \
                     .transpose(0, 2, 3, 1).reshape(n_sh, n_tb, rows, 1)

    total, count = pl.pallas_call(
        functools.partial(_fused_tagger_kernel, t_blk=t_blk, b_blk=b_blk,
                          hdim=H),
        out_shape=(jax.ShapeDtypeStruct((n_sh, 1, 1), jnp.float32),
                   jax.ShapeDtypeStruct((n_sh, 1, 1), jnp.float32)),
        grid_spec=pltpu.PrefetchScalarGridSpec(
            num_scalar_prefetch=0,
            grid=(n_sh, n_tb),
            in_specs=[
                pl.BlockSpec((1, rows, E), lambda s, t: (s, t, 0)),
                pl.BlockSpec((1, 1, rows, 1), lambda s, t: (s, t, 0, 0)),
                pl.BlockSpec((E, 4 * H), lambda s, t: (0, 0)),
                pl.BlockSpec((1, 4 * H), lambda s, t: (0, 0)),
                pl.BlockSpec((H, 4 * H), lambda s, t: (0, 0)),
                pl.BlockSpec((H, C), lambda s, t: (0, 0)),
                pl.BlockSpec((1, C), lambda s, t: (0, 0)),
            ],
            out_specs=[
                pl.BlockSpec((1, 1, 1), lambda s, t: (s, 0, 0)),
                pl.BlockSpec((1, 1, 1), lambda s, t: (s, 0, 0)),
            ],
            scratch_shapes=[
                pltpu.VMEM((b_blk, H), jnp.float32),
                pltpu.VMEM((b_blk, H), jnp.float32),
                pltpu.VMEM((t_blk * b_blk, H), jnp.bfloat16),
            ],
        ),
        compiler_params=pltpu.CompilerParams(
            dimension_semantics=("parallel", "arbitrary"),
            vmem_limit_bytes=64 * 1024 * 1024),
    )(emb_r, labels_r, w_ih_t.astype(jnp.bfloat16), b,
      w_hh_t.astype(jnp.bfloat16), w_cls_t.astype(jnp.bfloat16), b_cls)

    return jnp.sum(total) / jnp.sum(count)


# speedup vs baseline: 1.5147x; 1.0115x over previous
import functools

import jax
import jax.numpy as jnp
from jax.experimental import pallas as pl
from jax.experimental.pallas import tpu as pltpu

_IGNORE = -100


def _fused_tagger_kernel(emb_ref, labels_ref, w_ih_ref, b_ref, w_hh_ref,
                         w_cls_ref, b_cls_ref, total_ref, count_ref,
                         h_sc, c_sc, hbuf_sc, *, t_blk, b_blk, hdim):
    @pl.when(pl.program_id(1) == 0)
    def _init():
        h_sc[...] = jnp.zeros_like(h_sc)
        c_sc[...] = jnp.zeros_like(c_sc)
        total_ref[...] = jnp.zeros_like(total_ref)
        count_ref[...] = jnp.zeros_like(count_ref)

    gx = jnp.dot(emb_ref[0], w_ih_ref[...],
                 preferred_element_type=jnp.float32) + b_ref[...]

    whh = w_hh_ref[...]
    for t in range(t_blk):
        gates = gx[t * b_blk:(t + 1) * b_blk] + jnp.dot(
            h_sc[...].astype(jnp.bfloat16), whh,
            preferred_element_type=jnp.float32)
        i_g = jax.nn.sigmoid(gates[:, 0 * hdim:1 * hdim])
        f_g = jax.nn.sigmoid(gates[:, 1 * hdim:2 * hdim])
        g_g = jnp.tanh(gates[:, 2 * hdim:3 * hdim])
        o_g = jax.nn.sigmoid(gates[:, 3 * hdim:4 * hdim])
        c_new = f_g * c_sc[...] + i_g * g_g
        h_new = o_g * jnp.tanh(c_new)
        c_sc[...] = c_new
        h_sc[...] = h_new
        hbuf_sc[t * b_blk:(t + 1) * b_blk, :] = h_new.astype(jnp.bfloat16)

    logits = jnp.dot(hbuf_sc[...], w_cls_ref[...],
                     preferred_element_type=jnp.float32) + b_cls_ref[...]
    lab = labels_ref[0, 0]
    valid = lab != _IGNORE
    m = jnp.max(logits, axis=1, keepdims=True)
    lse = m + jnp.log(jnp.sum(jnp.exp(logits - m), axis=1, keepdims=True))
    cls = jax.lax.broadcasted_iota(jnp.int32, logits.shape, 1)
    safe = jnp.where(valid, lab, 0)
    picked = jnp.sum(jnp.where(cls == safe, logits, 0.0), axis=1,
                     keepdims=True)
    nll = jnp.where(valid, lse - picked, 0.0)
    total_ref[...] += jnp.sum(nll).reshape(1, 1, 1)
    count_ref[...] += jnp.sum(valid.astype(jnp.float32)).reshape(1, 1, 1)


def kernel(tokens, labels, embedding, w_ih_t, w_hh_t, b, w_cls_t, b_cls):
    B, T = tokens.shape
    E = embedding.shape[1]
    H = w_hh_t.shape[0]
    C = w_cls_t.shape[1]

    b_blk = B
    n_sh = B // b_blk
    t_blk = 32 if T % 32 == 0 else T
    n_tb = T // t_blk

    rows = t_blk * b_blk
    tokens_r = tokens.reshape(n_sh, b_blk, T).transpose(0, 2, 1) \
                     .reshape(n_sh, T * b_blk)
    emb_r = embedding[tokens_r].astype(jnp.bfloat16)
    labels_r = labels.reshape(n_sh, b_blk, n_tb, t_blk) \
                     .transpose(0, 2, 3, 1).reshape(n_sh, n_tb, rows, 1)

    total, count = pl.pallas_call(
        functools.partial(_fused_tagger_kernel, t_blk=t_blk, b_blk=b_blk,
                          hdim=H),
        out_shape=(jax.ShapeDtypeStruct((n_sh, 1, 1), jnp.float32),
                   jax.ShapeDtypeStruct((n_sh, 1, 1), jnp.float32)),
        grid_spec=pltpu.PrefetchScalarGridSpec(
            num_scalar_prefetch=0,
            grid=(n_sh, n_tb),
            in_specs=[
                pl.BlockSpec((1, rows, E), lambda s, t: (s, t, 0)),
                pl.BlockSpec((1, 1, rows, 1), lambda s, t: (s, t, 0, 0)),
                pl.BlockSpec((E, 4 * H), lambda s, t: (0, 0)),
                pl.BlockSpec((1, 4 * H), lambda s, t: (0, 0)),
                pl.BlockSpec((H, 4 * H), lambda s, t: (0, 0)),
                pl.BlockSpec((H, C), lambda s, t: (0, 0)),
                pl.BlockSpec((1, C), lambda s, t: (0, 0)),
            ],
            out_specs=[
                pl.BlockSpec((1, 1, 1), lambda s, t: (s, 0, 0)),
                pl.BlockSpec((1, 1, 1), lambda s, t: (s, 0, 0)),
            ],
            scratch_shapes=[
                pltpu.VMEM((b_blk, H), jnp.float32),
                pltpu.VMEM((b_blk, H), jnp.float32),
                pltpu.VMEM((t_blk * b_blk, H), jnp.bfloat16),
            ],
        ),
        compiler_params=pltpu.CompilerParams(
            dimension_semantics=("parallel", "arbitrary"),
            vmem_limit_bytes=64 * 1024 * 1024),
    )(emb_r, labels_r, w_ih_t.astype(jnp.bfloat16), b,
      w_hh_t.astype(jnp.bfloat16), w_cls_t.astype(jnp.bfloat16), b_cls)

    return jnp.sum(total) / jnp.sum(count)
